# baseline (speedup 1.0000x reference)
import functools
import math

import jax
import jax.numpy as jnp
from jax import lax
from jax.experimental import pallas as pl
from jax.experimental.pallas import tpu as pltpu
from jax.experimental.pallas import tpu_sc as plsc

D_MODEL = 64
SCALE = math.sqrt(D_MODEL)
NC = 2
NS = 16
NW = NC * NS
L = 16
NBLK = 128


def _embed(xt, table, n_tokens, seq):
    n_blocks = n_tokens // NBLK
    assert n_blocks == NW

    mesh = plsc.VectorSubcoreMesh(core_axis_name="c", subcore_axis_name="s")

    @functools.partial(
        pl.kernel,
        mesh=mesh,
        out_type=jax.ShapeDtypeStruct(
            (seq, D_MODEL // 8, n_blocks, 8, NBLK), jnp.float32
        ),
        compiler_params=pltpu.CompilerParams(
            use_tc_tiling_on_sc=False, needs_layout_passes=False
        ),
        scratch_types=[
            pltpu.VMEM((seq, NBLK), jnp.int32),
            pltpu.VMEM((2, NBLK, D_MODEL), jnp.float32),
            pltpu.VMEM((2, D_MODEL // 8, 8, NBLK), jnp.float32),
            [pltpu.SemaphoreType.DMA] * 2,
            [pltpu.SemaphoreType.DMA] * 2,
        ],
    )
    def k(x_hbm, table_hbm, out_hbm, idx_v, gbufs, obufs, gsems, ssems):
        wid = lax.axis_index("s") * NC + lax.axis_index("c")
        pltpu.sync_copy(x_hbm.at[:, pl.ds(wid * NBLK, NBLK)], idx_v)

        def gather_start(s, p):
            pltpu.async_copy(
                table_hbm.at[idx_v.at[s]], gbufs.at[p], gsems[p]
            )

        def gather_wait(p):
            pltpu.make_async_copy(
                table_hbm.at[idx_v.at[0]], gbufs.at[p], gsems[p]
            ).wait()

        def store_start(s, p):
            pltpu.async_copy(
                obufs.at[p], out_hbm.at[s, :, wid], ssems[p]
            )

        def store_wait(p):
            pltpu.make_async_copy(
                obufs.at[p], out_hbm.at[0, :, wid], ssems[p]
            ).wait()

        gather_start(0, 0)
        gather_start(1, 1)

        @pl.loop(0, seq, step=2)
        def superstep(s0):
            for p in range(2):
                s = s0 + p
                gather_wait(p)

                @pl.when(s >= 2)
                def _():
                    store_wait(p)

                rows = [
                    jax.lax.iota(jnp.int32, L) + (j * L) for j in range(NBLK // L)
                ]

                @plsc.parallel_loop(0, D_MODEL, unroll=2)
                def col(d):
                    cold = jnp.full((L,), d, jnp.int32)
                    dt = d // 8
                    db = d % 8
                    for j in range(NBLK // L):
                        v = plsc.load_gather(gbufs.at[p], [rows[j], cold])
                        obufs[p, dt, db, pl.ds(j * L, L)] = v * SCALE

                @pl.when(s + 2 < seq)
                def _():
                    gather_start(s + 2, p)

                store_start(s, p)

        for p in range(2):
            store_wait(p)

    return k(xt, table)


def kernel(x, table):
    n, seq = x.shape
    out5 = _embed(x.T, table, n, seq)
    return out5.transpose(2, 4, 0, 1, 3).reshape(n, seq, D_MODEL)


# speedup vs baseline: 1.0016x; 1.0016x over previous
import functools
import math

import jax
import jax.numpy as jnp
from jax import lax
from jax.experimental import pallas as pl
from jax.experimental.pallas import tpu as pltpu
from jax.experimental.pallas import tpu_sc as plsc

D_MODEL = 64
SCALE = math.sqrt(D_MODEL)
NC = 2
NS = 16
NW = NC * NS
L = 16
NBLK = 128
NB = 4


def _embed(xt, table, n_tokens, seq):
    n_blocks = n_tokens // NBLK
    assert n_blocks == NW and seq % NB == 0

    mesh = plsc.VectorSubcoreMesh(core_axis_name="c", subcore_axis_name="s")

    @functools.partial(
        pl.kernel,
        mesh=mesh,
        out_type=jax.ShapeDtypeStruct(
            (seq, D_MODEL // 8, n_blocks, 8, NBLK), jnp.float32
        ),
        compiler_params=pltpu.CompilerParams(
            use_tc_tiling_on_sc=False, needs_layout_passes=False
        ),
        scratch_types=[
            pltpu.VMEM((seq, NBLK), jnp.int32),
            pltpu.VMEM((NB, NBLK, D_MODEL), jnp.float32),
            pltpu.VMEM((NB, D_MODEL // 8, 8, NBLK), jnp.float32),
            [pltpu.SemaphoreType.DMA] * NB,
            [pltpu.SemaphoreType.DMA] * NB,
        ],
    )
    def k(x_hbm, tab_hbm, out_hbm, idx_v, gbufs, obufs, gsems, ssems):
        wid = lax.axis_index("s") * NC + lax.axis_index("c")
        pltpu.sync_copy(x_hbm.at[:, pl.ds(wid * NBLK, NBLK)], idx_v)

        def gather_start(s, p):
            pltpu.async_copy(
                tab_hbm.at[idx_v.at[s]], gbufs.at[p], gsems[p]
            )

        def gather_wait(p):
            pltpu.make_async_copy(
                tab_hbm.at[idx_v.at[0]], gbufs.at[p], gsems[p]
            ).wait()

        def store_start(s, p):
            pltpu.async_copy(
                obufs.at[p], out_hbm.at[s, :, wid], ssems[p]
            )

        def store_wait(p):
            pltpu.make_async_copy(
                obufs.at[p], out_hbm.at[0, :, wid], ssems[p]
            ).wait()

        gather_start(0, 0)
        gather_start(1, 1)

        rows = [jax.lax.iota(jnp.int32, L) + (j * L) for j in range(NBLK // L)]

        @pl.loop(0, seq, step=NB)
        def superstep(s0):
            for p in range(NB):
                s = s0 + p
                gather_wait(p)

                @pl.when(s >= NB)
                def _():
                    store_wait(p)

                @plsc.parallel_loop(0, D_MODEL, unroll=4)
                def col(d):
                    cold = jnp.full((L,), d, jnp.int32)
                    dt = d // 8
                    db = d % 8
                    for j in range(NBLK // L):
                        v = plsc.load_gather(gbufs.at[p], [rows[j], cold])
                        obufs[p, dt, db, pl.ds(j * L, L)] = v * SCALE

                @pl.when(s + 2 < seq)
                def _():
                    gather_start(s + 2, (p + 2) % NB)

                store_start(s, p)

        for p in range(NB):
            store_wait(p)

    return k(xt, table)


def kernel(x, table):
    n, seq = x.shape
    out5 = _embed(x.T, table, n, seq)
    return out5.transpose(2, 4, 0, 1, 3).reshape(n, seq, D_MODEL)


# speedup vs baseline: 1.6174x; 1.6148x over previous
import functools
import math

import jax
import jax.numpy as jnp
from jax import lax
from jax.experimental import pallas as pl
from jax.experimental.pallas import tpu as pltpu
from jax.experimental.pallas import tpu_sc as plsc

D_MODEL = 64
SCALE = math.sqrt(D_MODEL)
NC = 2
NS = 16
NW = NC * NS
L = 16
NBLK = 128
NB = 4


def _embed(xt, table, n_tokens, seq):
    n_blocks = n_tokens // NBLK
    assert n_blocks == NW and seq % NB == 0

    mesh = plsc.VectorSubcoreMesh(core_axis_name="c", subcore_axis_name="s")

    @functools.partial(
        pl.kernel,
        mesh=mesh,
        out_type=jax.ShapeDtypeStruct(
            (seq, D_MODEL // 8, n_blocks, 8, NBLK), jnp.float32
        ),
        compiler_params=pltpu.CompilerParams(
            use_tc_tiling_on_sc=False, needs_layout_passes=False
        ),
        scratch_types=[
            pltpu.VMEM((seq, NBLK), jnp.int32),
            pltpu.VMEM((NB, NBLK, D_MODEL), jnp.float32),
            pltpu.VMEM((NB, D_MODEL // 8, 8, NBLK + 1), jnp.float32),
            [pltpu.SemaphoreType.DMA] * NB,
            [pltpu.SemaphoreType.DMA] * NB,
        ],
    )
    def k(x_hbm, tab_hbm, out_hbm, idx_v, gbufs, obufs, gsems, ssems):
        wid = lax.axis_index("s") * NC + lax.axis_index("c")
        pltpu.sync_copy(x_hbm.at[:, pl.ds(wid * NBLK, NBLK)], idx_v)

        def gather_start(s, p):
            pltpu.async_copy(
                tab_hbm.at[idx_v.at[s]], gbufs.at[p], gsems[p]
            )

        def gather_wait(p):
            pltpu.make_async_copy(
                tab_hbm.at[idx_v.at[0]], gbufs.at[p], gsems[p]
            ).wait()

        def store_start(s, p):
            pltpu.async_copy(
                obufs.at[p, :, :, pl.ds(0, NBLK)],
                out_hbm.at[s, :, wid],
                ssems[p],
            )

        def store_wait(p):
            pltpu.make_async_copy(
                obufs.at[p, :, :, pl.ds(0, NBLK)],
                out_hbm.at[0, :, wid],
                ssems[p],
            ).wait()

        gather_start(0, 0)
        gather_start(1, 1)

        iota = jax.lax.iota(jnp.int32, L)
        dts = [(iota + q * L) >> 3 for q in range(D_MODEL // L)]
        dbs = [(iota + q * L) & 7 for q in range(D_MODEL // L)]

        @pl.loop(0, seq, step=NB)
        def superstep(s0):
            for p in range(NB):
                s = s0 + p
                gather_wait(p)

                @pl.when(s >= NB)
                def _():
                    store_wait(p)

                @plsc.parallel_loop(0, NBLK, unroll=4)
                def tok(t):
                    colt = jnp.full((L,), t, jnp.int32)
                    for q in range(D_MODEL // L):
                        v = gbufs[p, t, pl.ds(q * L, L)]
                        plsc.store_scatter(
                            obufs.at[p], [dts[q], dbs[q], colt], v * SCALE
                        )

                @pl.when(s + 2 < seq)
                def _():
                    gather_start(s + 2, (p + 2) % NB)

                store_start(s, p)

        for p in range(NB):
            store_wait(p)

    return k(xt, table)


def kernel(x, table):
    n, seq = x.shape
    out5 = _embed(x.T, table, n, seq)
    return out5.transpose(2, 4, 0, 1, 3).reshape(n, seq, D_MODEL)
